# baseline (speedup 1.0000x reference)
import functools

import jax
import jax.numpy as jnp
import numpy as np
from jax.experimental import pallas as pl
from jax.experimental.pallas import tpu as pltpu
from jax.experimental.pallas import tpu_sc as plsc

_TOP_K = 32
_IMAX = np.int32(2**31 - 1)

_THRESH = np.float32(2.25)
_KST = np.int32(np.float32(2.25).view(np.int32))
_NC = 2
_NS = 16
_NW = _NC * _NS
_LANES = 16
_CHUNK = 8192
_UNROLL = 4
_LCAP = 2048
_PCAP = 1024
_TCAP = _LANES * _LCAP
_T2 = 8

_T = 16
_NB = 16
_NBM = 16
_MAXT = 16


def _num_passes(width, t):
    w = width
    p = 0
    while w > 0:
        w //= t + 1
        p += 1
    return p


_P = _num_passes(2**32 - 1, _T)
_R2 = _num_passes(int(_IMAX) - int(_KST), _T2)


def _keys_of(x):
    b = jax.lax.bitcast_convert_type(x, jnp.int32)
    flip = jax.lax.shift_right_arithmetic(b, 31) & jnp.int32(2**31 - 1)
    return b ^ flip


def _sc_compact_kernel(x_hbm, val_hbm, idx_hbm, cnt_hbm,
                       vbuf_a, vbuf_b, val_l, idx_l, cnt_l, sem_a, sem_b, *,
                       nchunk):
    wid = jax.lax.axis_index("s") * _NC + jax.lax.axis_index("c")
    base = wid * (_CHUNK * nchunk)

    zero16 = jnp.zeros((_LANES,), jnp.float32)

    def _zinit(i, _):
        val_l[pl.ds(i * _LANES, _LANES)] = zero16
        return 0

    jax.lax.fori_loop(0, _TCAP // _LANES, _zinit, 0)

    lane_base = jax.lax.iota(jnp.int32, _LANES) * _LCAP
    pos = jnp.zeros((_LANES,), jnp.int32)
    iv = jax.lax.iota(jnp.int32, _LANES) + base

    bufs = [vbuf_a, vbuf_b]

    def _make_inner(cur):
        def _inner(i, carry):
            pos, iv = carry
            for u in range(_UNROLL):
                v = bufs[cur][pl.ds((i * _UNROLL + u) * _LANES, _LANES)]
                m = (v >= _THRESH) & (pos < _LCAP)
                tgt = lane_base + pos
                plsc.store_scatter(val_l, [tgt], v, mask=m)
                plsc.store_scatter(idx_l, [tgt], iv, mask=m)
                pos = pos + m.astype(jnp.int32)
                iv = iv + _LANES
            return pos, iv
        return _inner

    sems = [sem_a, sem_b]
    copies = [None, None]
    copies[0] = pltpu.async_copy(x_hbm.at[pl.ds(base, _CHUNK)],
                                 bufs[0], sems[0])
    for c in range(nchunk):
        cur = c % 2
        if c + 1 < nchunk:
            nxt = (c + 1) % 2
            copies[nxt] = pltpu.async_copy(
                x_hbm.at[pl.ds(base + (c + 1) * _CHUNK, _CHUNK)],
                bufs[nxt], sems[nxt])
        copies[cur].wait()
        pos, iv = jax.lax.fori_loop(0, _CHUNK // (_LANES * _UNROLL),
                                    _make_inner(cur), (pos, iv))

    cnt_l[...] = pos
    pltpu.sync_copy(val_l, val_hbm.at[pl.ds(wid * _TCAP, _TCAP)])
    pltpu.sync_copy(idx_l, idx_hbm.at[pl.ds(wid * _TCAP, _TCAP)])
    pltpu.sync_copy(cnt_l, cnt_hbm.at[pl.ds(wid * _LANES, _LANES)])


def _select_kernel(val_ref, idx_ref, cnt_ref, meta_ref, *, k_total):
    vb = jax.lax.bitcast_convert_type(val_ref[...], jnp.int32)
    cnt = cnt_ref[...]

    n_cand = jnp.sum(cnt)
    ok = jnp.all(cnt <= _PCAP) & (n_cand >= k_total)

    lo = jnp.int32(_KST)
    hi = jnp.int32(_IMAX)
    for _ in range(_R2):
        step = jax.lax.div(hi - lo, jnp.int32(_T2 + 1)) + jnp.int32(1)
        ts = [lo + step * jnp.int32(i + 1) - jnp.int32(1) for i in range(_T2)]
        cs = [jnp.sum((vb > t).astype(jnp.int32)) for t in ts]
        for i in range(_T2):
            lo = jnp.where(cs[i] >= k_total, ts[i] + jnp.int32(1), lo)
            hi = jnp.where(cs[i] < k_total, jnp.minimum(hi, ts[i]), hi)
    kappa = lo

    tie = vb == kappa
    m = jnp.sum(tie.astype(jnp.int32))
    c_ge = jnp.sum((vb >= kappa).astype(jnp.int32))
    r = k_total - (c_ge - m)
    ok = ok & (m <= _MAXT)

    idx = idx_ref[...]
    cand0 = jnp.where(tie, idx, _IMAX)
    cutoff = jnp.int32(-1)
    last = jnp.int32(-1)
    for j in range(_MAXT):
        nxt = jnp.min(jnp.where(cand0 > last, cand0, _IMAX))
        cutoff = jnp.where(jnp.int32(j) == r - 1, nxt, cutoff)
        last = nxt

    meta_ref[0] = jnp.where(ok, jnp.int32(1), jnp.int32(0))
    meta_ref[1] = kappa
    meta_ref[2] = cutoff
    meta_ref[3] = n_cand


def _fastmask_kernel(meta_ref, x_ref, o_ref, *, rows, d):
    b = pl.program_id(0)
    kappa = meta_ref[1]
    cutoff = meta_ref[2]
    x = x_ref[...]
    xb = jax.lax.bitcast_convert_type(x, jnp.int32)
    tie = xb == kappa
    m_b = jnp.sum(tie.astype(jnp.int32))

    @pl.when(m_b == 0)
    def _simple():
        o_ref[...] = jnp.where(xb >= kappa, x, jnp.float32(0))

    @pl.when(m_b > 0)
    def _withties():
        r_iota = jax.lax.broadcasted_iota(jnp.int32, (rows, d), 0)
        c_iota = jax.lax.broadcasted_iota(jnp.int32, (rows, d), 1)
        flat = (r_iota + b * rows) * d + c_iota
        keep = (xb > kappa) | (tie & (flat <= cutoff))
        o_ref[...] = jnp.where(keep, x, jnp.float32(0))


def _bisect_kernel(x_ref, kappa_ref, counts_ref, state_ref, *, k_total, nb, t,
                   n_passes):
    p = pl.program_id(0)
    b = pl.program_id(1)

    @pl.when((p == 0) & (b == 0))
    def _init():
        state_ref[0] = jnp.int32(-(2**31))
        state_ref[1] = jnp.int32(2**31 - 1)
        for i in range(t):
            counts_ref[i] = jnp.int32(0)

    lo = state_ref[0]
    hi = state_ref[1]
    step0 = jnp.int32(2**32 // (t + 1) + 1)
    stepg = jax.lax.div(hi - lo, jnp.int32(t + 1)) + jnp.int32(1)
    step = jnp.where(p == 0, step0, stepg)
    ts = [lo + step * jnp.int32(i + 1) - jnp.int32(1) for i in range(t)]

    keys = _keys_of(x_ref[...])
    for i in range(t):
        c = jnp.sum((keys > ts[i]).astype(jnp.int32))
        counts_ref[i] = counts_ref[i] + c

    @pl.when(b == nb - 1)
    def _update():
        new_lo = lo
        new_hi = hi
        for i in range(t):
            ci = counts_ref[i]
            new_lo = jnp.where(ci >= k_total, ts[i] + jnp.int32(1), new_lo)
            new_hi = jnp.where(ci < k_total, jnp.minimum(new_hi, ts[i]), new_hi)
        state_ref[0] = new_lo
        state_ref[1] = new_hi
        for i in range(t):
            counts_ref[i] = jnp.int32(0)

        @pl.when(p == n_passes - 1)
        def _fin():
            kappa_ref[0] = new_lo


def _ties_kernel(kappa_ref, x_ref, meta_ref, *, maxt, rows, d):
    b = pl.program_id(0)

    @pl.when(b == 0)
    def _init():
        for i in range(maxt + 2):
            meta_ref[i] = jnp.int32(0)

    kappa = kappa_ref[0]
    keys = _keys_of(x_ref[...])
    ge = keys >= kappa
    meta_ref[maxt] = meta_ref[maxt] + jnp.sum(ge.astype(jnp.int32))
    tie = keys == kappa
    m_b = jnp.sum(tie.astype(jnp.int32))

    @pl.when(m_b > 0)
    def _extract():
        r_iota = jax.lax.broadcasted_iota(jnp.int32, (rows, d), 0)
        c_iota = jax.lax.broadcasted_iota(jnp.int32, (rows, d), 1)
        flat = (r_iota + b * rows) * d + c_iota
        cand = jnp.where(tie, flat, _IMAX)
        last = jnp.int32(-1)
        for _ in range(maxt):
            nxt = jnp.min(jnp.where(cand > last, cand, _IMAX))
            found = nxt != _IMAX
            pos = meta_ref[maxt + 1]

            @pl.when(found & (pos < maxt))
            def _store():
                meta_ref[pos] = nxt

            meta_ref[maxt + 1] = jnp.where(found, pos + 1, pos)
            last = jnp.where(found, nxt, last)


def _mask_kernel(kappa_ref, meta_ref, x_ref, o_ref, *, k_total, maxt, rows, d):
    b = pl.program_id(0)
    kappa = kappa_ref[0]
    c_ge = meta_ref[maxt]
    n_ties = meta_ref[maxt + 1]
    c_gt = c_ge - n_ties
    r = k_total - c_gt
    ridx = jnp.clip(r - 1, 0, maxt - 1)
    cutoff = jnp.where(r <= 0, jnp.int32(-1), meta_ref[ridx])

    x = x_ref[...]
    keys = _keys_of(x)
    tie = keys == kappa
    m_b = jnp.sum(tie.astype(jnp.int32))

    @pl.when(m_b == 0)
    def _simple():
        o_ref[...] = jnp.where(keys >= kappa, x, jnp.float32(0))

    @pl.when(m_b > 0)
    def _withties():
        r_iota = jax.lax.broadcasted_iota(jnp.int32, (rows, d), 0)
        c_iota = jax.lax.broadcasted_iota(jnp.int32, (rows, d), 1)
        flat = (r_iota + b * rows) * d + c_iota
        keep = (keys > kappa) | (tie & (flat <= cutoff))
        o_ref[...] = jnp.where(keep, x, jnp.float32(0))


def _slow_path(x, k_total):
    bsz, d = x.shape
    rows = bsz // _NBM
    kappa = pl.pallas_call(
        functools.partial(_bisect_kernel, k_total=k_total, nb=_NB, t=_T,
                          n_passes=_P),
        grid=(_P, _NB),
        in_specs=[pl.BlockSpec((bsz, d // _NB), lambda p, b: (0, b))],
        out_specs=pl.BlockSpec(memory_space=pltpu.SMEM),
        out_shape=jax.ShapeDtypeStruct((1,), jnp.int32),
        scratch_shapes=[pltpu.SMEM((_T,), jnp.int32),
                        pltpu.SMEM((2,), jnp.int32)],
    )(x)

    meta = pl.pallas_call(
        functools.partial(_ties_kernel, maxt=_MAXT, rows=rows, d=d),
        grid=(_NBM,),
        in_specs=[pl.BlockSpec(memory_space=pltpu.SMEM),
                  pl.BlockSpec((rows, d), lambda b: (b, 0))],
        out_specs=pl.BlockSpec(memory_space=pltpu.SMEM),
        out_shape=jax.ShapeDtypeStruct((_MAXT + 2,), jnp.int32),
    )(kappa, x)

    return pl.pallas_call(
        functools.partial(_mask_kernel, k_total=k_total, maxt=_MAXT, rows=rows,
                          d=d),
        grid=(_NBM,),
        in_specs=[pl.BlockSpec(memory_space=pltpu.SMEM),
                  pl.BlockSpec(memory_space=pltpu.SMEM),
                  pl.BlockSpec((rows, d), lambda b: (b, 0))],
        out_specs=pl.BlockSpec((rows, d), lambda b: (b, 0)),
        out_shape=jax.ShapeDtypeStruct((bsz, d), x.dtype),
    )(kappa, meta, x)


@jax.jit
def kernel(x):
    bsz, d = x.shape
    n = bsz * d
    k_total = min(_TOP_K * bsz, n)
    nchunk = n // (_NW * _CHUNK)

    mesh = plsc.VectorSubcoreMesh(core_axis_name="c", subcore_axis_name="s")
    compact = pl.kernel(
        functools.partial(_sc_compact_kernel, nchunk=nchunk),
        out_type=(jax.ShapeDtypeStruct((_NW * _TCAP,), jnp.float32),
                  jax.ShapeDtypeStruct((_NW * _TCAP,), jnp.int32),
                  jax.ShapeDtypeStruct((_NW * _LANES,), jnp.int32)),
        mesh=mesh,
        scratch_types=(pltpu.VMEM((_CHUNK,), jnp.float32),
                       pltpu.VMEM((_CHUNK,), jnp.float32),
                       pltpu.VMEM((_TCAP,), jnp.float32),
                       pltpu.VMEM((_TCAP,), jnp.int32),
                       pltpu.VMEM((_LANES,), jnp.int32),
                       pltpu.SemaphoreType.DMA,
                       pltpu.SemaphoreType.DMA),
        compiler_params=pltpu.CompilerParams(needs_layout_passes=False),
    )
    cval, cidx, ccnt = compact(x.reshape(-1))

    meta = pl.pallas_call(
        functools.partial(_select_kernel, k_total=k_total),
        grid=(1,),
        in_specs=[pl.BlockSpec((_NW * _LANES, _PCAP), lambda i: (0, 0)),
                  pl.BlockSpec((_NW * _LANES, _PCAP), lambda i: (0, 0)),
                  pl.BlockSpec((4, 128), lambda i: (0, 0))],
        out_specs=pl.BlockSpec(memory_space=pltpu.SMEM),
        out_shape=jax.ShapeDtypeStruct((4,), jnp.int32),
    )(cval.reshape(_NW * _LANES, _LCAP), cidx.reshape(_NW * _LANES, _LCAP),
      ccnt.reshape(4, 128))

    rows = bsz // _NBM

    def _fast(x, meta):
        return pl.pallas_call(
            functools.partial(_fastmask_kernel, rows=rows, d=d),
            grid=(_NBM,),
            in_specs=[pl.BlockSpec(memory_space=pltpu.SMEM),
                      pl.BlockSpec((rows, d), lambda b: (b, 0))],
            out_specs=pl.BlockSpec((rows, d), lambda b: (b, 0)),
            out_shape=jax.ShapeDtypeStruct((bsz, d), x.dtype),
        )(meta, x)

    return jax.lax.cond(meta[0] == 1,
                        lambda: _fast(x, meta),
                        lambda: _slow_path(x, k_total))


# speedup vs baseline: 1.0043x; 1.0043x over previous
import functools

import jax
import jax.numpy as jnp
import numpy as np
from jax.experimental import pallas as pl
from jax.experimental.pallas import tpu as pltpu
from jax.experimental.pallas import tpu_sc as plsc

_TOP_K = 32
_IMAX = np.int32(2**31 - 1)

_THRESH = np.float32(2.25)
_KST = np.int32(np.float32(2.25).view(np.int32))
_NC = 2
_NS = 16
_NW = _NC * _NS
_LANES = 16
_CHUNK = 16384
_NBUF = 3
_UNROLL = 4
_LCAP = 2048
_PCAP = 1024
_TCAP = _LANES * _LCAP
_T2 = 8

_T = 16
_NB = 16
_NBM = 16
_MAXT = 16


def _num_passes(width, t):
    w = width
    p = 0
    while w > 0:
        w //= t + 1
        p += 1
    return p


_P = _num_passes(2**32 - 1, _T)
_R2 = _num_passes(int(_IMAX) - int(_KST), _T2)


def _keys_of(x):
    b = jax.lax.bitcast_convert_type(x, jnp.int32)
    flip = jax.lax.shift_right_arithmetic(b, 31) & jnp.int32(2**31 - 1)
    return b ^ flip


def _sc_compact_kernel(x_hbm, val_hbm, idx_hbm, cnt_hbm,
                       vbuf_a, vbuf_b, vbuf_c, val_l, idx_l, cnt_l,
                       sem_a, sem_b, sem_c, *, nchunk):
    wid = jax.lax.axis_index("s") * _NC + jax.lax.axis_index("c")
    base = wid * (_CHUNK * nchunk)

    zero16 = jnp.zeros((_LANES,), jnp.float32)

    def _zinit(i, _):
        val_l[pl.ds(i * _LANES, _LANES)] = zero16
        return 0

    jax.lax.fori_loop(0, _TCAP // _LANES, _zinit, 0)

    lane_base = jax.lax.iota(jnp.int32, _LANES) * _LCAP
    pos = jnp.zeros((_LANES,), jnp.int32)
    iv = jax.lax.iota(jnp.int32, _LANES) + base

    bufs = [vbuf_a, vbuf_b, vbuf_c]

    def _make_inner(cur):
        def _inner(i, carry):
            pos, iv = carry
            for u in range(_UNROLL):
                v = bufs[cur][pl.ds((i * _UNROLL + u) * _LANES, _LANES)]
                m = (v >= _THRESH) & (pos < _LCAP)
                tgt = lane_base + pos
                plsc.store_scatter(val_l, [tgt], v, mask=m)
                plsc.store_scatter(idx_l, [tgt], iv, mask=m)
                pos = pos + m.astype(jnp.int32)
                iv = iv + _LANES
            return pos, iv
        return _inner

    sems = [sem_a, sem_b, sem_c]
    copies = [None] * _NBUF
    for c in range(min(_NBUF, nchunk)):
        copies[c] = pltpu.async_copy(
            x_hbm.at[pl.ds(base + c * _CHUNK, _CHUNK)], bufs[c], sems[c])
    for c in range(nchunk):
        cur = c % _NBUF
        copies[cur].wait()
        pos, iv = jax.lax.fori_loop(0, _CHUNK // (_LANES * _UNROLL),
                                    _make_inner(cur), (pos, iv))
        if c + _NBUF < nchunk:
            copies[cur] = pltpu.async_copy(
                x_hbm.at[pl.ds(base + (c + _NBUF) * _CHUNK, _CHUNK)],
                bufs[cur], sems[cur])

    cnt_l[...] = pos
    pltpu.sync_copy(val_l, val_hbm.at[pl.ds(wid * _TCAP, _TCAP)])
    pltpu.sync_copy(idx_l, idx_hbm.at[pl.ds(wid * _TCAP, _TCAP)])
    pltpu.sync_copy(cnt_l, cnt_hbm.at[pl.ds(wid * _LANES, _LANES)])


def _select_kernel(val_ref, idx_ref, cnt_ref, meta_ref, *, k_total):
    vb = jax.lax.bitcast_convert_type(val_ref[...], jnp.int32)
    cnt = cnt_ref[...]

    n_cand = jnp.sum(cnt)
    ok = jnp.all(cnt <= _PCAP) & (n_cand >= k_total)

    lo = jnp.int32(_KST)
    hi = jnp.int32(_IMAX)
    for _ in range(_R2):
        step = jax.lax.div(hi - lo, jnp.int32(_T2 + 1)) + jnp.int32(1)
        ts = [lo + step * jnp.int32(i + 1) - jnp.int32(1) for i in range(_T2)]
        cs = [jnp.sum((vb > t).astype(jnp.int32)) for t in ts]
        for i in range(_T2):
            lo = jnp.where(cs[i] >= k_total, ts[i] + jnp.int32(1), lo)
            hi = jnp.where(cs[i] < k_total, jnp.minimum(hi, ts[i]), hi)
    kappa = lo

    tie = vb == kappa
    m = jnp.sum(tie.astype(jnp.int32))
    c_ge = jnp.sum((vb >= kappa).astype(jnp.int32))
    r = k_total - (c_ge - m)
    ok = ok & (m <= _MAXT)

    idx = idx_ref[...]
    cand0 = jnp.where(tie, idx, _IMAX)
    cutoff = jnp.int32(-1)
    last = jnp.int32(-1)
    for j in range(_MAXT):
        nxt = jnp.min(jnp.where(cand0 > last, cand0, _IMAX))
        cutoff = jnp.where(jnp.int32(j) == r - 1, nxt, cutoff)
        last = nxt

    meta_ref[0] = jnp.where(ok, jnp.int32(1), jnp.int32(0))
    meta_ref[1] = kappa
    meta_ref[2] = cutoff
    meta_ref[3] = n_cand


def _fastmask_kernel(meta_ref, x_ref, o_ref, *, rows, d):
    b = pl.program_id(0)
    kappa = meta_ref[1]
    cutoff = meta_ref[2]
    x = x_ref[...]
    xb = jax.lax.bitcast_convert_type(x, jnp.int32)
    tie = xb == kappa
    m_b = jnp.sum(tie.astype(jnp.int32))

    @pl.when(m_b == 0)
    def _simple():
        o_ref[...] = jnp.where(xb >= kappa, x, jnp.float32(0))

    @pl.when(m_b > 0)
    def _withties():
        r_iota = jax.lax.broadcasted_iota(jnp.int32, (rows, d), 0)
        c_iota = jax.lax.broadcasted_iota(jnp.int32, (rows, d), 1)
        flat = (r_iota + b * rows) * d + c_iota
        keep = (xb > kappa) | (tie & (flat <= cutoff))
        o_ref[...] = jnp.where(keep, x, jnp.float32(0))


def _bisect_kernel(x_ref, kappa_ref, counts_ref, state_ref, *, k_total, nb, t,
                   n_passes):
    p = pl.program_id(0)
    b = pl.program_id(1)

    @pl.when((p == 0) & (b == 0))
    def _init():
        state_ref[0] = jnp.int32(-(2**31))
        state_ref[1] = jnp.int32(2**31 - 1)
        for i in range(t):
            counts_ref[i] = jnp.int32(0)

    lo = state_ref[0]
    hi = state_ref[1]
    step0 = jnp.int32(2**32 // (t + 1) + 1)
    stepg = jax.lax.div(hi - lo, jnp.int32(t + 1)) + jnp.int32(1)
    step = jnp.where(p == 0, step0, stepg)
    ts = [lo + step * jnp.int32(i + 1) - jnp.int32(1) for i in range(t)]

    keys = _keys_of(x_ref[...])
    for i in range(t):
        c = jnp.sum((keys > ts[i]).astype(jnp.int32))
        counts_ref[i] = counts_ref[i] + c

    @pl.when(b == nb - 1)
    def _update():
        new_lo = lo
        new_hi = hi
        for i in range(t):
            ci = counts_ref[i]
            new_lo = jnp.where(ci >= k_total, ts[i] + jnp.int32(1), new_lo)
            new_hi = jnp.where(ci < k_total, jnp.minimum(new_hi, ts[i]), new_hi)
        state_ref[0] = new_lo
        state_ref[1] = new_hi
        for i in range(t):
            counts_ref[i] = jnp.int32(0)

        @pl.when(p == n_passes - 1)
        def _fin():
            kappa_ref[0] = new_lo


def _ties_kernel(kappa_ref, x_ref, meta_ref, *, maxt, rows, d):
    b = pl.program_id(0)

    @pl.when(b == 0)
    def _init():
        for i in range(maxt + 2):
            meta_ref[i] = jnp.int32(0)

    kappa = kappa_ref[0]
    keys = _keys_of(x_ref[...])
    ge = keys >= kappa
    meta_ref[maxt] = meta_ref[maxt] + jnp.sum(ge.astype(jnp.int32))
    tie = keys == kappa
    m_b = jnp.sum(tie.astype(jnp.int32))

    @pl.when(m_b > 0)
    def _extract():
        r_iota = jax.lax.broadcasted_iota(jnp.int32, (rows, d), 0)
        c_iota = jax.lax.broadcasted_iota(jnp.int32, (rows, d), 1)
        flat = (r_iota + b * rows) * d + c_iota
        cand = jnp.where(tie, flat, _IMAX)
        last = jnp.int32(-1)
        for _ in range(maxt):
            nxt = jnp.min(jnp.where(cand > last, cand, _IMAX))
            found = nxt != _IMAX
            pos = meta_ref[maxt + 1]

            @pl.when(found & (pos < maxt))
            def _store():
                meta_ref[pos] = nxt

            meta_ref[maxt + 1] = jnp.where(found, pos + 1, pos)
            last = jnp.where(found, nxt, last)


def _mask_kernel(kappa_ref, meta_ref, x_ref, o_ref, *, k_total, maxt, rows, d):
    b = pl.program_id(0)
    kappa = kappa_ref[0]
    c_ge = meta_ref[maxt]
    n_ties = meta_ref[maxt + 1]
    c_gt = c_ge - n_ties
    r = k_total - c_gt
    ridx = jnp.clip(r - 1, 0, maxt - 1)
    cutoff = jnp.where(r <= 0, jnp.int32(-1), meta_ref[ridx])

    x = x_ref[...]
    keys = _keys_of(x)
    tie = keys == kappa
    m_b = jnp.sum(tie.astype(jnp.int32))

    @pl.when(m_b == 0)
    def _simple():
        o_ref[...] = jnp.where(keys >= kappa, x, jnp.float32(0))

    @pl.when(m_b > 0)
    def _withties():
        r_iota = jax.lax.broadcasted_iota(jnp.int32, (rows, d), 0)
        c_iota = jax.lax.broadcasted_iota(jnp.int32, (rows, d), 1)
        flat = (r_iota + b * rows) * d + c_iota
        keep = (keys > kappa) | (tie & (flat <= cutoff))
        o_ref[...] = jnp.where(keep, x, jnp.float32(0))


def _slow_path(x, k_total):
    bsz, d = x.shape
    rows = bsz // _NBM
    kappa = pl.pallas_call(
        functools.partial(_bisect_kernel, k_total=k_total, nb=_NB, t=_T,
                          n_passes=_P),
        grid=(_P, _NB),
        in_specs=[pl.BlockSpec((bsz, d // _NB), lambda p, b: (0, b))],
        out_specs=pl.BlockSpec(memory_space=pltpu.SMEM),
        out_shape=jax.ShapeDtypeStruct((1,), jnp.int32),
        scratch_shapes=[pltpu.SMEM((_T,), jnp.int32),
                        pltpu.SMEM((2,), jnp.int32)],
    )(x)

    meta = pl.pallas_call(
        functools.partial(_ties_kernel, maxt=_MAXT, rows=rows, d=d),
        grid=(_NBM,),
        in_specs=[pl.BlockSpec(memory_space=pltpu.SMEM),
                  pl.BlockSpec((rows, d), lambda b: (b, 0))],
        out_specs=pl.BlockSpec(memory_space=pltpu.SMEM),
        out_shape=jax.ShapeDtypeStruct((_MAXT + 2,), jnp.int32),
    )(kappa, x)

    return pl.pallas_call(
        functools.partial(_mask_kernel, k_total=k_total, maxt=_MAXT, rows=rows,
                          d=d),
        grid=(_NBM,),
        in_specs=[pl.BlockSpec(memory_space=pltpu.SMEM),
                  pl.BlockSpec(memory_space=pltpu.SMEM),
                  pl.BlockSpec((rows, d), lambda b: (b, 0))],
        out_specs=pl.BlockSpec((rows, d), lambda b: (b, 0)),
        out_shape=jax.ShapeDtypeStruct((bsz, d), x.dtype),
    )(kappa, meta, x)


@jax.jit
def kernel(x):
    bsz, d = x.shape
    n = bsz * d
    k_total = min(_TOP_K * bsz, n)
    nchunk = n // (_NW * _CHUNK)

    mesh = plsc.VectorSubcoreMesh(core_axis_name="c", subcore_axis_name="s")
    compact = pl.kernel(
        functools.partial(_sc_compact_kernel, nchunk=nchunk),
        out_type=(jax.ShapeDtypeStruct((_NW * _TCAP,), jnp.float32),
                  jax.ShapeDtypeStruct((_NW * _TCAP,), jnp.int32),
                  jax.ShapeDtypeStruct((_NW * _LANES,), jnp.int32)),
        mesh=mesh,
        scratch_types=(pltpu.VMEM((_CHUNK,), jnp.float32),
                       pltpu.VMEM((_CHUNK,), jnp.float32),
                       pltpu.VMEM((_CHUNK,), jnp.float32),
                       pltpu.VMEM((_TCAP,), jnp.float32),
                       pltpu.VMEM((_TCAP,), jnp.int32),
                       pltpu.VMEM((_LANES,), jnp.int32),
                       pltpu.SemaphoreType.DMA,
                       pltpu.SemaphoreType.DMA,
                       pltpu.SemaphoreType.DMA),
        compiler_params=pltpu.CompilerParams(needs_layout_passes=False),
    )
    cval, cidx, ccnt = compact(x.reshape(-1))

    meta = pl.pallas_call(
        functools.partial(_select_kernel, k_total=k_total),
        grid=(1,),
        in_specs=[pl.BlockSpec((_NW * _LANES, _PCAP), lambda i: (0, 0)),
                  pl.BlockSpec((_NW * _LANES, _PCAP), lambda i: (0, 0)),
                  pl.BlockSpec((4, 128), lambda i: (0, 0))],
        out_specs=pl.BlockSpec(memory_space=pltpu.SMEM),
        out_shape=jax.ShapeDtypeStruct((4,), jnp.int32),
    )(cval.reshape(_NW * _LANES, _LCAP), cidx.reshape(_NW * _LANES, _LCAP),
      ccnt.reshape(4, 128))

    rows = bsz // _NBM

    def _fast(x, meta):
        return pl.pallas_call(
            functools.partial(_fastmask_kernel, rows=rows, d=d),
            grid=(_NBM,),
            in_specs=[pl.BlockSpec(memory_space=pltpu.SMEM),
                      pl.BlockSpec((rows, d), lambda b: (b, 0))],
            out_specs=pl.BlockSpec((rows, d), lambda b: (b, 0)),
            out_shape=jax.ShapeDtypeStruct((bsz, d), x.dtype),
        )(meta, x)

    return jax.lax.cond(meta[0] == 1,
                        lambda: _fast(x, meta),
                        lambda: _slow_path(x, k_total))


# speedup vs baseline: 1.0472x; 1.0427x over previous
import functools

import jax
import jax.numpy as jnp
import numpy as np
from jax.experimental import pallas as pl
from jax.experimental.pallas import tpu as pltpu
from jax.experimental.pallas import tpu_sc as plsc

_TOP_K = 32
_IMAX = np.int32(2**31 - 1)

_THRESH = np.float32(2.25)
_KST = np.int32(np.float32(2.25).view(np.int32))
_NC = 2
_NS = 16
_NW = _NC * _NS
_LANES = 16
_CHUNK = 16384
_NBUF = 3
_UNROLL = 4
_LCAP = 2048
_PCAP = 1024
_TCAP = _LANES * _LCAP
_T2 = 8

_T = 16
_NB = 16
_NBM = 16
_MAXT = 16


def _num_passes(width, t):
    w = width
    p = 0
    while w > 0:
        w //= t + 1
        p += 1
    return p


_P = _num_passes(2**32 - 1, _T)
_R2 = _num_passes(int(_IMAX) - int(_KST), _T2)


def _keys_of(x):
    b = jax.lax.bitcast_convert_type(x, jnp.int32)
    flip = jax.lax.shift_right_arithmetic(b, 31) & jnp.int32(2**31 - 1)
    return b ^ flip


def _sc_compact_kernel(x_hbm, val_hbm, idx_hbm, cnt_hbm,
                       vbuf_a, vbuf_b, vbuf_c, val_l, idx_l, cnt_l,
                       sem_a, sem_b, sem_c, *, nchunk):
    wid = jax.lax.axis_index("s") * _NC + jax.lax.axis_index("c")
    base = wid * (_CHUNK * nchunk)

    zero16 = jnp.zeros((_LANES,), jnp.float32)

    def _zinit(i, _):
        val_l[pl.ds(i * _LANES, _LANES)] = zero16
        return 0

    jax.lax.fori_loop(0, _TCAP // _LANES, _zinit, 0)

    lane_base = jax.lax.iota(jnp.int32, _LANES) * _LCAP
    pos = jnp.zeros((_LANES,), jnp.int32)
    iv = jax.lax.iota(jnp.int32, _LANES) + base

    bufs = [vbuf_a, vbuf_b, vbuf_c]

    def _make_inner(cur):
        def _inner(i, carry):
            pos, iv = carry
            for u in range(_UNROLL):
                v = bufs[cur][pl.ds((i * _UNROLL + u) * _LANES, _LANES)]
                m = v >= _THRESH
                tgt = lane_base + jnp.minimum(pos, _LCAP - 1)
                plsc.store_scatter(val_l, [tgt], v, mask=m)
                plsc.store_scatter(idx_l, [tgt], iv, mask=m)
                pos = pos + m.astype(jnp.int32)
                iv = iv + _LANES
            return pos, iv
        return _inner

    sems = [sem_a, sem_b, sem_c]
    copies = [None] * _NBUF
    for c in range(min(_NBUF, nchunk)):
        copies[c] = pltpu.async_copy(
            x_hbm.at[pl.ds(base + c * _CHUNK, _CHUNK)], bufs[c], sems[c])
    for c in range(nchunk):
        cur = c % _NBUF
        copies[cur].wait()
        pos, iv = jax.lax.fori_loop(0, _CHUNK // (_LANES * _UNROLL),
                                    _make_inner(cur), (pos, iv))
        if c + _NBUF < nchunk:
            copies[cur] = pltpu.async_copy(
                x_hbm.at[pl.ds(base + (c + _NBUF) * _CHUNK, _CHUNK)],
                bufs[cur], sems[cur])

    cnt_l[...] = pos
    pltpu.sync_copy(val_l, val_hbm.at[pl.ds(wid * _TCAP, _TCAP)])
    pltpu.sync_copy(idx_l, idx_hbm.at[pl.ds(wid * _TCAP, _TCAP)])
    pltpu.sync_copy(cnt_l, cnt_hbm.at[pl.ds(wid * _LANES, _LANES)])


def _select_kernel(val_ref, idx_ref, cnt_ref, meta_ref, *, k_total):
    vb = jax.lax.bitcast_convert_type(val_ref[...], jnp.int32)
    cnt = cnt_ref[...]

    n_cand = jnp.sum(cnt)
    ok = jnp.all(cnt <= _PCAP) & (n_cand >= k_total)

    lo = jnp.int32(_KST)
    hi = jnp.int32(_IMAX)
    for _ in range(_R2):
        step = jax.lax.div(hi - lo, jnp.int32(_T2 + 1)) + jnp.int32(1)
        ts = [lo + step * jnp.int32(i + 1) - jnp.int32(1) for i in range(_T2)]
        cs = [jnp.sum((vb > t).astype(jnp.int32)) for t in ts]
        for i in range(_T2):
            lo = jnp.where(cs[i] >= k_total, ts[i] + jnp.int32(1), lo)
            hi = jnp.where(cs[i] < k_total, jnp.minimum(hi, ts[i]), hi)
    kappa = lo

    tie = vb == kappa
    m = jnp.sum(tie.astype(jnp.int32))
    c_ge = jnp.sum((vb >= kappa).astype(jnp.int32))
    r = k_total - (c_ge - m)
    ok = ok & (m <= _MAXT)

    idx = idx_ref[...]
    cand0 = jnp.where(tie, idx, _IMAX)
    cutoff = jnp.int32(-1)
    last = jnp.int32(-1)
    for j in range(_MAXT):
        nxt = jnp.min(jnp.where(cand0 > last, cand0, _IMAX))
        cutoff = jnp.where(jnp.int32(j) == r - 1, nxt, cutoff)
        last = nxt

    meta_ref[0] = jnp.where(ok, jnp.int32(1), jnp.int32(0))
    meta_ref[1] = kappa
    meta_ref[2] = cutoff
    meta_ref[3] = n_cand


def _fastmask_kernel(meta_ref, x_ref, o_ref, *, rows, d):
    b = pl.program_id(0)
    kappa = meta_ref[1]
    cutoff = meta_ref[2]
    x = x_ref[...]
    xb = jax.lax.bitcast_convert_type(x, jnp.int32)
    tie = xb == kappa
    m_b = jnp.sum(tie.astype(jnp.int32))

    @pl.when(m_b == 0)
    def _simple():
        o_ref[...] = jnp.where(xb >= kappa, x, jnp.float32(0))

    @pl.when(m_b > 0)
    def _withties():
        r_iota = jax.lax.broadcasted_iota(jnp.int32, (rows, d), 0)
        c_iota = jax.lax.broadcasted_iota(jnp.int32, (rows, d), 1)
        flat = (r_iota + b * rows) * d + c_iota
        keep = (xb > kappa) | (tie & (flat <= cutoff))
        o_ref[...] = jnp.where(keep, x, jnp.float32(0))


def _bisect_kernel(x_ref, kappa_ref, counts_ref, state_ref, *, k_total, nb, t,
                   n_passes):
    p = pl.program_id(0)
    b = pl.program_id(1)

    @pl.when((p == 0) & (b == 0))
    def _init():
        state_ref[0] = jnp.int32(-(2**31))
        state_ref[1] = jnp.int32(2**31 - 1)
        for i in range(t):
            counts_ref[i] = jnp.int32(0)

    lo = state_ref[0]
    hi = state_ref[1]
    step0 = jnp.int32(2**32 // (t + 1) + 1)
    stepg = jax.lax.div(hi - lo, jnp.int32(t + 1)) + jnp.int32(1)
    step = jnp.where(p == 0, step0, stepg)
    ts = [lo + step * jnp.int32(i + 1) - jnp.int32(1) for i in range(t)]

    keys = _keys_of(x_ref[...])
    for i in range(t):
        c = jnp.sum((keys > ts[i]).astype(jnp.int32))
        counts_ref[i] = counts_ref[i] + c

    @pl.when(b == nb - 1)
    def _update():
        new_lo = lo
        new_hi = hi
        for i in range(t):
            ci = counts_ref[i]
            new_lo = jnp.where(ci >= k_total, ts[i] + jnp.int32(1), new_lo)
            new_hi = jnp.where(ci < k_total, jnp.minimum(new_hi, ts[i]), new_hi)
        state_ref[0] = new_lo
        state_ref[1] = new_hi
        for i in range(t):
            counts_ref[i] = jnp.int32(0)

        @pl.when(p == n_passes - 1)
        def _fin():
            kappa_ref[0] = new_lo


def _ties_kernel(kappa_ref, x_ref, meta_ref, *, maxt, rows, d):
    b = pl.program_id(0)

    @pl.when(b == 0)
    def _init():
        for i in range(maxt + 2):
            meta_ref[i] = jnp.int32(0)

    kappa = kappa_ref[0]
    keys = _keys_of(x_ref[...])
    ge = keys >= kappa
    meta_ref[maxt] = meta_ref[maxt] + jnp.sum(ge.astype(jnp.int32))
    tie = keys == kappa
    m_b = jnp.sum(tie.astype(jnp.int32))

    @pl.when(m_b > 0)
    def _extract():
        r_iota = jax.lax.broadcasted_iota(jnp.int32, (rows, d), 0)
        c_iota = jax.lax.broadcasted_iota(jnp.int32, (rows, d), 1)
        flat = (r_iota + b * rows) * d + c_iota
        cand = jnp.where(tie, flat, _IMAX)
        last = jnp.int32(-1)
        for _ in range(maxt):
            nxt = jnp.min(jnp.where(cand > last, cand, _IMAX))
            found = nxt != _IMAX
            pos = meta_ref[maxt + 1]

            @pl.when(found & (pos < maxt))
            def _store():
                meta_ref[pos] = nxt

            meta_ref[maxt + 1] = jnp.where(found, pos + 1, pos)
            last = jnp.where(found, nxt, last)


def _mask_kernel(kappa_ref, meta_ref, x_ref, o_ref, *, k_total, maxt, rows, d):
    b = pl.program_id(0)
    kappa = kappa_ref[0]
    c_ge = meta_ref[maxt]
    n_ties = meta_ref[maxt + 1]
    c_gt = c_ge - n_ties
    r = k_total - c_gt
    ridx = jnp.clip(r - 1, 0, maxt - 1)
    cutoff = jnp.where(r <= 0, jnp.int32(-1), meta_ref[ridx])

    x = x_ref[...]
    keys = _keys_of(x)
    tie = keys == kappa
    m_b = jnp.sum(tie.astype(jnp.int32))

    @pl.when(m_b == 0)
    def _simple():
        o_ref[...] = jnp.where(keys >= kappa, x, jnp.float32(0))

    @pl.when(m_b > 0)
    def _withties():
        r_iota = jax.lax.broadcasted_iota(jnp.int32, (rows, d), 0)
        c_iota = jax.lax.broadcasted_iota(jnp.int32, (rows, d), 1)
        flat = (r_iota + b * rows) * d + c_iota
        keep = (keys > kappa) | (tie & (flat <= cutoff))
        o_ref[...] = jnp.where(keep, x, jnp.float32(0))


def _slow_path(x, k_total):
    bsz, d = x.shape
    rows = bsz // _NBM
    kappa = pl.pallas_call(
        functools.partial(_bisect_kernel, k_total=k_total, nb=_NB, t=_T,
                          n_passes=_P),
        grid=(_P, _NB),
        in_specs=[pl.BlockSpec((bsz, d // _NB), lambda p, b: (0, b))],
        out_specs=pl.BlockSpec(memory_space=pltpu.SMEM),
        out_shape=jax.ShapeDtypeStruct((1,), jnp.int32),
        scratch_shapes=[pltpu.SMEM((_T,), jnp.int32),
                        pltpu.SMEM((2,), jnp.int32)],
    )(x)

    meta = pl.pallas_call(
        functools.partial(_ties_kernel, maxt=_MAXT, rows=rows, d=d),
        grid=(_NBM,),
        in_specs=[pl.BlockSpec(memory_space=pltpu.SMEM),
                  pl.BlockSpec((rows, d), lambda b: (b, 0))],
        out_specs=pl.BlockSpec(memory_space=pltpu.SMEM),
        out_shape=jax.ShapeDtypeStruct((_MAXT + 2,), jnp.int32),
    )(kappa, x)

    return pl.pallas_call(
        functools.partial(_mask_kernel, k_total=k_total, maxt=_MAXT, rows=rows,
                          d=d),
        grid=(_NBM,),
        in_specs=[pl.BlockSpec(memory_space=pltpu.SMEM),
                  pl.BlockSpec(memory_space=pltpu.SMEM),
                  pl.BlockSpec((rows, d), lambda b: (b, 0))],
        out_specs=pl.BlockSpec((rows, d), lambda b: (b, 0)),
        out_shape=jax.ShapeDtypeStruct((bsz, d), x.dtype),
    )(kappa, meta, x)


@jax.jit
def kernel(x):
    bsz, d = x.shape
    n = bsz * d
    k_total = min(_TOP_K * bsz, n)
    nchunk = n // (_NW * _CHUNK)

    mesh = plsc.VectorSubcoreMesh(core_axis_name="c", subcore_axis_name="s")
    compact = pl.kernel(
        functools.partial(_sc_compact_kernel, nchunk=nchunk),
        out_type=(jax.ShapeDtypeStruct((_NW * _TCAP,), jnp.float32),
                  jax.ShapeDtypeStruct((_NW * _TCAP,), jnp.int32),
                  jax.ShapeDtypeStruct((_NW * _LANES,), jnp.int32)),
        mesh=mesh,
        scratch_types=(pltpu.VMEM((_CHUNK,), jnp.float32),
                       pltpu.VMEM((_CHUNK,), jnp.float32),
                       pltpu.VMEM((_CHUNK,), jnp.float32),
                       pltpu.VMEM((_TCAP,), jnp.float32),
                       pltpu.VMEM((_TCAP,), jnp.int32),
                       pltpu.VMEM((_LANES,), jnp.int32),
                       pltpu.SemaphoreType.DMA,
                       pltpu.SemaphoreType.DMA,
                       pltpu.SemaphoreType.DMA),
        compiler_params=pltpu.CompilerParams(needs_layout_passes=False),
    )
    cval, cidx, ccnt = compact(x.reshape(-1))

    meta = pl.pallas_call(
        functools.partial(_select_kernel, k_total=k_total),
        grid=(1,),
        in_specs=[pl.BlockSpec((_NW * _LANES, _PCAP), lambda i: (0, 0)),
                  pl.BlockSpec((_NW * _LANES, _PCAP), lambda i: (0, 0)),
                  pl.BlockSpec((4, 128), lambda i: (0, 0))],
        out_specs=pl.BlockSpec(memory_space=pltpu.SMEM),
        out_shape=jax.ShapeDtypeStruct((4,), jnp.int32),
    )(cval.reshape(_NW * _LANES, _LCAP), cidx.reshape(_NW * _LANES, _LCAP),
      ccnt.reshape(4, 128))

    rows = bsz // _NBM

    def _fast(x, meta):
        return pl.pallas_call(
            functools.partial(_fastmask_kernel, rows=rows, d=d),
            grid=(_NBM,),
            in_specs=[pl.BlockSpec(memory_space=pltpu.SMEM),
                      pl.BlockSpec((rows, d), lambda b: (b, 0))],
            out_specs=pl.BlockSpec((rows, d), lambda b: (b, 0)),
            out_shape=jax.ShapeDtypeStruct((bsz, d), x.dtype),
        )(meta, x)

    return jax.lax.cond(meta[0] == 1,
                        lambda: _fast(x, meta),
                        lambda: _slow_path(x, k_total))


# speedup vs baseline: 1.9846x; 1.8952x over previous
import functools

import jax
import jax.numpy as jnp
import numpy as np
from jax.experimental import pallas as pl
from jax.experimental.pallas import tpu as pltpu
from jax.experimental.pallas import tpu_sc as plsc

_TOP_K = 32
_IMAX = np.int32(2**31 - 1)

_THRESH = np.float32(2.25)
_KST = np.int32(np.float32(2.25).view(np.int32))
_NC = 2
_NS = 16
_NW = _NC * _NS
_LANES = 16
_CHUNK = 16384
_NBUF = 3
_UNROLL = 4
_LCAP = 2048
_PCAP = 1024
_TCAP = _LANES * _LCAP
_T2 = 8

_T = 16
_NB = 16
_NBM = 16
_MAXT = 16


def _num_passes(width, t):
    w = width
    p = 0
    while w > 0:
        w //= t + 1
        p += 1
    return p


_P = _num_passes(2**32 - 1, _T)
_R2 = _num_passes(int(_IMAX) - int(_KST), _T2)


def _keys_of(x):
    b = jax.lax.bitcast_convert_type(x, jnp.int32)
    flip = jax.lax.shift_right_arithmetic(b, 31) & jnp.int32(2**31 - 1)
    return b ^ flip


def _sc_compact_kernel(x_hbm, val_hbm, idx_hbm, cnt_hbm,
                       vbuf_a, vbuf_b, vbuf_c, val_l, idx_l, cnt_l,
                       sem_a, sem_b, sem_c, *, nchunk):
    wid = jax.lax.axis_index("s") * _NC + jax.lax.axis_index("c")
    base = wid * (_CHUNK * nchunk)

    zero16 = jnp.zeros((_LANES,), jnp.float32)

    @plsc.parallel_loop(0, _TCAP // _LANES, unroll=8)
    def _zinit(i):
        val_l[pl.ds(i * _LANES, _LANES)] = zero16

    lane_base = jax.lax.iota(jnp.int32, _LANES) * _LCAP
    pos = jnp.zeros((_LANES,), jnp.int32)
    iv = jax.lax.iota(jnp.int32, _LANES) + base

    bufs = [vbuf_a, vbuf_b, vbuf_c]

    def _run_chunk(cur, pos, iv):
        def _inner(i, carry):
            pos, iv = carry
            v = bufs[cur][pl.ds(i * _LANES, _LANES)]
            m = v >= _THRESH
            tgt = lane_base + jnp.minimum(pos, _LCAP - 1)
            plsc.store_scatter(val_l, [tgt], v, mask=m)
            plsc.store_scatter(idx_l, [tgt], iv, mask=m)
            pos = pos + m.astype(jnp.int32)
            iv = iv + _LANES
            return pos, iv

        return plsc.parallel_loop(0, _CHUNK // _LANES, unroll=_UNROLL,
                                  carry=(pos, iv))(_inner)

    sems = [sem_a, sem_b, sem_c]
    copies = [None] * _NBUF
    for c in range(min(_NBUF, nchunk)):
        copies[c] = pltpu.async_copy(
            x_hbm.at[pl.ds(base + c * _CHUNK, _CHUNK)], bufs[c], sems[c])
    for c in range(nchunk):
        cur = c % _NBUF
        copies[cur].wait()
        pos, iv = _run_chunk(cur, pos, iv)
        if c + _NBUF < nchunk:
            copies[cur] = pltpu.async_copy(
                x_hbm.at[pl.ds(base + (c + _NBUF) * _CHUNK, _CHUNK)],
                bufs[cur], sems[cur])

    cnt_l[...] = pos
    pltpu.sync_copy(val_l, val_hbm.at[pl.ds(wid * _TCAP, _TCAP)])
    pltpu.sync_copy(idx_l, idx_hbm.at[pl.ds(wid * _TCAP, _TCAP)])
    pltpu.sync_copy(cnt_l, cnt_hbm.at[pl.ds(wid * _LANES, _LANES)])


def _select_kernel(val_ref, idx_ref, cnt_ref, meta_ref, *, k_total):
    vb = jax.lax.bitcast_convert_type(val_ref[...], jnp.int32)
    cnt = cnt_ref[...]

    n_cand = jnp.sum(cnt)
    ok = jnp.all(cnt <= _PCAP) & (n_cand >= k_total)

    lo = jnp.int32(_KST)
    hi = jnp.int32(_IMAX)
    for _ in range(_R2):
        step = jax.lax.div(hi - lo, jnp.int32(_T2 + 1)) + jnp.int32(1)
        ts = [lo + step * jnp.int32(i + 1) - jnp.int32(1) for i in range(_T2)]
        cs = [jnp.sum((vb > t).astype(jnp.int32)) for t in ts]
        for i in range(_T2):
            lo = jnp.where(cs[i] >= k_total, ts[i] + jnp.int32(1), lo)
            hi = jnp.where(cs[i] < k_total, jnp.minimum(hi, ts[i]), hi)
    kappa = lo

    tie = vb == kappa
    m = jnp.sum(tie.astype(jnp.int32))
    c_ge = jnp.sum((vb >= kappa).astype(jnp.int32))
    r = k_total - (c_ge - m)
    ok = ok & (m <= _MAXT)

    idx = idx_ref[...]
    cand0 = jnp.where(tie, idx, _IMAX)
    cutoff = jnp.int32(-1)
    last = jnp.int32(-1)
    for j in range(_MAXT):
        nxt = jnp.min(jnp.where(cand0 > last, cand0, _IMAX))
        cutoff = jnp.where(jnp.int32(j) == r - 1, nxt, cutoff)
        last = nxt

    meta_ref[0] = jnp.where(ok, jnp.int32(1), jnp.int32(0))
    meta_ref[1] = kappa
    meta_ref[2] = cutoff
    meta_ref[3] = n_cand


def _fastmask_kernel(meta_ref, x_ref, o_ref, *, rows, d):
    b = pl.program_id(0)
    kappa = meta_ref[1]
    cutoff = meta_ref[2]
    x = x_ref[...]
    xb = jax.lax.bitcast_convert_type(x, jnp.int32)
    tie = xb == kappa
    m_b = jnp.sum(tie.astype(jnp.int32))

    @pl.when(m_b == 0)
    def _simple():
        o_ref[...] = jnp.where(xb >= kappa, x, jnp.float32(0))

    @pl.when(m_b > 0)
    def _withties():
        r_iota = jax.lax.broadcasted_iota(jnp.int32, (rows, d), 0)
        c_iota = jax.lax.broadcasted_iota(jnp.int32, (rows, d), 1)
        flat = (r_iota + b * rows) * d + c_iota
        keep = (xb > kappa) | (tie & (flat <= cutoff))
        o_ref[...] = jnp.where(keep, x, jnp.float32(0))


def _bisect_kernel(x_ref, kappa_ref, counts_ref, state_ref, *, k_total, nb, t,
                   n_passes):
    p = pl.program_id(0)
    b = pl.program_id(1)

    @pl.when((p == 0) & (b == 0))
    def _init():
        state_ref[0] = jnp.int32(-(2**31))
        state_ref[1] = jnp.int32(2**31 - 1)
        for i in range(t):
            counts_ref[i] = jnp.int32(0)

    lo = state_ref[0]
    hi = state_ref[1]
    step0 = jnp.int32(2**32 // (t + 1) + 1)
    stepg = jax.lax.div(hi - lo, jnp.int32(t + 1)) + jnp.int32(1)
    step = jnp.where(p == 0, step0, stepg)
    ts = [lo + step * jnp.int32(i + 1) - jnp.int32(1) for i in range(t)]

    keys = _keys_of(x_ref[...])
    for i in range(t):
        c = jnp.sum((keys > ts[i]).astype(jnp.int32))
        counts_ref[i] = counts_ref[i] + c

    @pl.when(b == nb - 1)
    def _update():
        new_lo = lo
        new_hi = hi
        for i in range(t):
            ci = counts_ref[i]
            new_lo = jnp.where(ci >= k_total, ts[i] + jnp.int32(1), new_lo)
            new_hi = jnp.where(ci < k_total, jnp.minimum(new_hi, ts[i]), new_hi)
        state_ref[0] = new_lo
        state_ref[1] = new_hi
        for i in range(t):
            counts_ref[i] = jnp.int32(0)

        @pl.when(p == n_passes - 1)
        def _fin():
            kappa_ref[0] = new_lo


def _ties_kernel(kappa_ref, x_ref, meta_ref, *, maxt, rows, d):
    b = pl.program_id(0)

    @pl.when(b == 0)
    def _init():
        for i in range(maxt + 2):
            meta_ref[i] = jnp.int32(0)

    kappa = kappa_ref[0]
    keys = _keys_of(x_ref[...])
    ge = keys >= kappa
    meta_ref[maxt] = meta_ref[maxt] + jnp.sum(ge.astype(jnp.int32))
    tie = keys == kappa
    m_b = jnp.sum(tie.astype(jnp.int32))

    @pl.when(m_b > 0)
    def _extract():
        r_iota = jax.lax.broadcasted_iota(jnp.int32, (rows, d), 0)
        c_iota = jax.lax.broadcasted_iota(jnp.int32, (rows, d), 1)
        flat = (r_iota + b * rows) * d + c_iota
        cand = jnp.where(tie, flat, _IMAX)
        last = jnp.int32(-1)
        for _ in range(maxt):
            nxt = jnp.min(jnp.where(cand > last, cand, _IMAX))
            found = nxt != _IMAX
            pos = meta_ref[maxt + 1]

            @pl.when(found & (pos < maxt))
            def _store():
                meta_ref[pos] = nxt

            meta_ref[maxt + 1] = jnp.where(found, pos + 1, pos)
            last = jnp.where(found, nxt, last)


def _mask_kernel(kappa_ref, meta_ref, x_ref, o_ref, *, k_total, maxt, rows, d):
    b = pl.program_id(0)
    kappa = kappa_ref[0]
    c_ge = meta_ref[maxt]
    n_ties = meta_ref[maxt + 1]
    c_gt = c_ge - n_ties
    r = k_total - c_gt
    ridx = jnp.clip(r - 1, 0, maxt - 1)
    cutoff = jnp.where(r <= 0, jnp.int32(-1), meta_ref[ridx])

    x = x_ref[...]
    keys = _keys_of(x)
    tie = keys == kappa
    m_b = jnp.sum(tie.astype(jnp.int32))

    @pl.when(m_b == 0)
    def _simple():
        o_ref[...] = jnp.where(keys >= kappa, x, jnp.float32(0))

    @pl.when(m_b > 0)
    def _withties():
        r_iota = jax.lax.broadcasted_iota(jnp.int32, (rows, d), 0)
        c_iota = jax.lax.broadcasted_iota(jnp.int32, (rows, d), 1)
        flat = (r_iota + b * rows) * d + c_iota
        keep = (keys > kappa) | (tie & (flat <= cutoff))
        o_ref[...] = jnp.where(keep, x, jnp.float32(0))


def _slow_path(x, k_total):
    bsz, d = x.shape
    rows = bsz // _NBM
    kappa = pl.pallas_call(
        functools.partial(_bisect_kernel, k_total=k_total, nb=_NB, t=_T,
                          n_passes=_P),
        grid=(_P, _NB),
        in_specs=[pl.BlockSpec((bsz, d // _NB), lambda p, b: (0, b))],
        out_specs=pl.BlockSpec(memory_space=pltpu.SMEM),
        out_shape=jax.ShapeDtypeStruct((1,), jnp.int32),
        scratch_shapes=[pltpu.SMEM((_T,), jnp.int32),
                        pltpu.SMEM((2,), jnp.int32)],
    )(x)

    meta = pl.pallas_call(
        functools.partial(_ties_kernel, maxt=_MAXT, rows=rows, d=d),
        grid=(_NBM,),
        in_specs=[pl.BlockSpec(memory_space=pltpu.SMEM),
                  pl.BlockSpec((rows, d), lambda b: (b, 0))],
        out_specs=pl.BlockSpec(memory_space=pltpu.SMEM),
        out_shape=jax.ShapeDtypeStruct((_MAXT + 2,), jnp.int32),
    )(kappa, x)

    return pl.pallas_call(
        functools.partial(_mask_kernel, k_total=k_total, maxt=_MAXT, rows=rows,
                          d=d),
        grid=(_NBM,),
        in_specs=[pl.BlockSpec(memory_space=pltpu.SMEM),
                  pl.BlockSpec(memory_space=pltpu.SMEM),
                  pl.BlockSpec((rows, d), lambda b: (b, 0))],
        out_specs=pl.BlockSpec((rows, d), lambda b: (b, 0)),
        out_shape=jax.ShapeDtypeStruct((bsz, d), x.dtype),
    )(kappa, meta, x)


@jax.jit
def kernel(x):
    bsz, d = x.shape
    n = bsz * d
    k_total = min(_TOP_K * bsz, n)
    nchunk = n // (_NW * _CHUNK)

    mesh = plsc.VectorSubcoreMesh(core_axis_name="c", subcore_axis_name="s")
    compact = pl.kernel(
        functools.partial(_sc_compact_kernel, nchunk=nchunk),
        out_type=(jax.ShapeDtypeStruct((_NW * _TCAP,), jnp.float32),
                  jax.ShapeDtypeStruct((_NW * _TCAP,), jnp.int32),
                  jax.ShapeDtypeStruct((_NW * _LANES,), jnp.int32)),
        mesh=mesh,
        scratch_types=(pltpu.VMEM((_CHUNK,), jnp.float32),
                       pltpu.VMEM((_CHUNK,), jnp.float32),
                       pltpu.VMEM((_CHUNK,), jnp.float32),
                       pltpu.VMEM((_TCAP,), jnp.float32),
                       pltpu.VMEM((_TCAP,), jnp.int32),
                       pltpu.VMEM((_LANES,), jnp.int32),
                       pltpu.SemaphoreType.DMA,
                       pltpu.SemaphoreType.DMA,
                       pltpu.SemaphoreType.DMA),
        compiler_params=pltpu.CompilerParams(needs_layout_passes=False),
    )
    cval, cidx, ccnt = compact(x.reshape(-1))

    meta = pl.pallas_call(
        functools.partial(_select_kernel, k_total=k_total),
        grid=(1,),
        in_specs=[pl.BlockSpec((_NW * _LANES, _PCAP), lambda i: (0, 0)),
                  pl.BlockSpec((_NW * _LANES, _PCAP), lambda i: (0, 0)),
                  pl.BlockSpec((4, 128), lambda i: (0, 0))],
        out_specs=pl.BlockSpec(memory_space=pltpu.SMEM),
        out_shape=jax.ShapeDtypeStruct((4,), jnp.int32),
    )(cval.reshape(_NW * _LANES, _LCAP), cidx.reshape(_NW * _LANES, _LCAP),
      ccnt.reshape(4, 128))

    rows = bsz // _NBM

    def _fast(x, meta):
        return pl.pallas_call(
            functools.partial(_fastmask_kernel, rows=rows, d=d),
            grid=(_NBM,),
            in_specs=[pl.BlockSpec(memory_space=pltpu.SMEM),
                      pl.BlockSpec((rows, d), lambda b: (b, 0))],
            out_specs=pl.BlockSpec((rows, d), lambda b: (b, 0)),
            out_shape=jax.ShapeDtypeStruct((bsz, d), x.dtype),
        )(meta, x)

    return jax.lax.cond(meta[0] == 1,
                        lambda: _fast(x, meta),
                        lambda: _slow_path(x, k_total))


# speedup vs baseline: 2.5383x; 1.2790x over previous
import functools

import jax
import jax.numpy as jnp
import numpy as np
from jax.experimental import pallas as pl
from jax.experimental.pallas import tpu as pltpu
from jax.experimental.pallas import tpu_sc as plsc

_TOP_K = 32
_IMAX = np.int32(2**31 - 1)

_THRESH = np.float32(2.25)
_KST = np.int32(np.float32(2.25).view(np.int32))
_NC = 2
_NS = 16
_NW = _NC * _NS
_LANES = 16
_CHUNK = 16384
_NBUF = 3
_UNROLL = 4
_LCAP = 2048
_PCAP = 1024
_TCAP = _LANES * _LCAP
_T2 = 8

_T = 16
_NB = 16
_NBM = 16
_MAXT = 16


def _num_passes(width, t):
    w = width
    p = 0
    while w > 0:
        w //= t + 1
        p += 1
    return p


_P = _num_passes(2**32 - 1, _T)
_R2 = _num_passes(int(_IMAX) - int(_KST), _T2)


def _keys_of(x):
    b = jax.lax.bitcast_convert_type(x, jnp.int32)
    flip = jax.lax.shift_right_arithmetic(b, 31) & jnp.int32(2**31 - 1)
    return b ^ flip


def _sc_compact_kernel(x_hbm, val_hbm, idx_hbm, cnt_hbm,
                       vbuf_a, vbuf_b, vbuf_c, val_l, idx_l, cnt_l,
                       sem_a, sem_b, sem_c, *, nchunk):
    wid = jax.lax.axis_index("s") * _NC + jax.lax.axis_index("c")
    base = wid * (_CHUNK * nchunk)

    zero16 = jnp.zeros((_LANES,), jnp.float32)

    @plsc.parallel_loop(0, _TCAP // _LANES, unroll=8)
    def _zinit(i):
        val_l[pl.ds(i * _LANES, _LANES)] = zero16

    lane_base = jax.lax.iota(jnp.int32, _LANES) * _LCAP
    pos = jnp.zeros((_LANES,), jnp.int32)
    iv = jax.lax.iota(jnp.int32, _LANES) + base

    bufs = [vbuf_a, vbuf_b, vbuf_c]

    def _run_chunk(cur, pos, iv):
        def _inner(i, carry):
            pos, iv = carry
            v = bufs[cur][0, pl.ds(i * _LANES, _LANES)]
            m = v >= _THRESH
            tgt = lane_base + jnp.minimum(pos, _LCAP - 1)
            plsc.store_scatter(val_l, [tgt], v, mask=m)
            plsc.store_scatter(idx_l, [tgt], iv, mask=m)
            pos = pos + m.astype(jnp.int32)
            iv = iv + _LANES
            return pos, iv

        return plsc.parallel_loop(0, _CHUNK // _LANES, unroll=_UNROLL,
                                  carry=(pos, iv))(_inner)

    row0 = wid * nchunk
    sems = [sem_a, sem_b, sem_c]
    copies = [None] * _NBUF
    for c in range(min(_NBUF, nchunk)):
        copies[c] = pltpu.async_copy(
            x_hbm.at[pl.ds(row0 + c, 1), :], bufs[c], sems[c])
    for c in range(nchunk):
        cur = c % _NBUF
        copies[cur].wait()
        pos, iv = _run_chunk(cur, pos, iv)
        if c + _NBUF < nchunk:
            copies[cur] = pltpu.async_copy(
                x_hbm.at[pl.ds(row0 + c + _NBUF, 1), :],
                bufs[cur], sems[cur])

    cnt_l[...] = pos
    pltpu.sync_copy(val_l, val_hbm.at[pl.ds(wid * _TCAP, _TCAP)])
    pltpu.sync_copy(idx_l, idx_hbm.at[pl.ds(wid * _TCAP, _TCAP)])
    pltpu.sync_copy(cnt_l, cnt_hbm.at[pl.ds(wid * _LANES, _LANES)])


def _select_kernel(val_ref, idx_ref, cnt_ref, meta_ref, *, k_total):
    vb = jax.lax.bitcast_convert_type(val_ref[...], jnp.int32)
    cnt = cnt_ref[...]

    n_cand = jnp.sum(cnt)
    ok = jnp.all(cnt <= _PCAP) & (n_cand >= k_total)

    lo = jnp.int32(_KST)
    hi = jnp.int32(_IMAX)
    for _ in range(_R2):
        step = jax.lax.div(hi - lo, jnp.int32(_T2 + 1)) + jnp.int32(1)
        ts = [lo + step * jnp.int32(i + 1) - jnp.int32(1) for i in range(_T2)]
        cs = [jnp.sum((vb > t).astype(jnp.int32)) for t in ts]
        for i in range(_T2):
            lo = jnp.where(cs[i] >= k_total, ts[i] + jnp.int32(1), lo)
            hi = jnp.where(cs[i] < k_total, jnp.minimum(hi, ts[i]), hi)
    kappa = lo

    tie = vb == kappa
    m = jnp.sum(tie.astype(jnp.int32))
    c_ge = jnp.sum((vb >= kappa).astype(jnp.int32))
    r = k_total - (c_ge - m)
    ok = ok & (m <= _MAXT)

    idx = idx_ref[...]
    cand0 = jnp.where(tie, idx, _IMAX)
    cutoff = jnp.int32(-1)
    last = jnp.int32(-1)
    for j in range(_MAXT):
        nxt = jnp.min(jnp.where(cand0 > last, cand0, _IMAX))
        cutoff = jnp.where(jnp.int32(j) == r - 1, nxt, cutoff)
        last = nxt

    meta_ref[0] = jnp.where(ok, jnp.int32(1), jnp.int32(0))
    meta_ref[1] = kappa
    meta_ref[2] = cutoff
    meta_ref[3] = n_cand


def _fastmask_kernel(meta_ref, x_ref, o_ref, *, rows, d):
    b = pl.program_id(0)
    kappa = meta_ref[1]
    cutoff = meta_ref[2]
    x = x_ref[...]
    xb = jax.lax.bitcast_convert_type(x, jnp.int32)
    tie = xb == kappa
    m_b = jnp.sum(tie.astype(jnp.int32))

    @pl.when(m_b == 0)
    def _simple():
        o_ref[...] = jnp.where(xb >= kappa, x, jnp.float32(0))

    @pl.when(m_b > 0)
    def _withties():
        r_iota = jax.lax.broadcasted_iota(jnp.int32, (rows, d), 0)
        c_iota = jax.lax.broadcasted_iota(jnp.int32, (rows, d), 1)
        flat = (r_iota + b * rows) * d + c_iota
        keep = (xb > kappa) | (tie & (flat <= cutoff))
        o_ref[...] = jnp.where(keep, x, jnp.float32(0))


def _bisect_kernel(x_ref, kappa_ref, counts_ref, state_ref, *, k_total, nb, t,
                   n_passes):
    p = pl.program_id(0)
    b = pl.program_id(1)

    @pl.when((p == 0) & (b == 0))
    def _init():
        state_ref[0] = jnp.int32(-(2**31))
        state_ref[1] = jnp.int32(2**31 - 1)
        for i in range(t):
            counts_ref[i] = jnp.int32(0)

    lo = state_ref[0]
    hi = state_ref[1]
    step0 = jnp.int32(2**32 // (t + 1) + 1)
    stepg = jax.lax.div(hi - lo, jnp.int32(t + 1)) + jnp.int32(1)
    step = jnp.where(p == 0, step0, stepg)
    ts = [lo + step * jnp.int32(i + 1) - jnp.int32(1) for i in range(t)]

    keys = _keys_of(x_ref[...])
    for i in range(t):
        c = jnp.sum((keys > ts[i]).astype(jnp.int32))
        counts_ref[i] = counts_ref[i] + c

    @pl.when(b == nb - 1)
    def _update():
        new_lo = lo
        new_hi = hi
        for i in range(t):
            ci = counts_ref[i]
            new_lo = jnp.where(ci >= k_total, ts[i] + jnp.int32(1), new_lo)
            new_hi = jnp.where(ci < k_total, jnp.minimum(new_hi, ts[i]), new_hi)
        state_ref[0] = new_lo
        state_ref[1] = new_hi
        for i in range(t):
            counts_ref[i] = jnp.int32(0)

        @pl.when(p == n_passes - 1)
        def _fin():
            kappa_ref[0] = new_lo


def _ties_kernel(kappa_ref, x_ref, meta_ref, *, maxt, rows, d):
    b = pl.program_id(0)

    @pl.when(b == 0)
    def _init():
        for i in range(maxt + 2):
            meta_ref[i] = jnp.int32(0)

    kappa = kappa_ref[0]
    keys = _keys_of(x_ref[...])
    ge = keys >= kappa
    meta_ref[maxt] = meta_ref[maxt] + jnp.sum(ge.astype(jnp.int32))
    tie = keys == kappa
    m_b = jnp.sum(tie.astype(jnp.int32))

    @pl.when(m_b > 0)
    def _extract():
        r_iota = jax.lax.broadcasted_iota(jnp.int32, (rows, d), 0)
        c_iota = jax.lax.broadcasted_iota(jnp.int32, (rows, d), 1)
        flat = (r_iota + b * rows) * d + c_iota
        cand = jnp.where(tie, flat, _IMAX)
        last = jnp.int32(-1)
        for _ in range(maxt):
            nxt = jnp.min(jnp.where(cand > last, cand, _IMAX))
            found = nxt != _IMAX
            pos = meta_ref[maxt + 1]

            @pl.when(found & (pos < maxt))
            def _store():
                meta_ref[pos] = nxt

            meta_ref[maxt + 1] = jnp.where(found, pos + 1, pos)
            last = jnp.where(found, nxt, last)


def _mask_kernel(kappa_ref, meta_ref, x_ref, o_ref, *, k_total, maxt, rows, d):
    b = pl.program_id(0)
    kappa = kappa_ref[0]
    c_ge = meta_ref[maxt]
    n_ties = meta_ref[maxt + 1]
    c_gt = c_ge - n_ties
    r = k_total - c_gt
    ridx = jnp.clip(r - 1, 0, maxt - 1)
    cutoff = jnp.where(r <= 0, jnp.int32(-1), meta_ref[ridx])

    x = x_ref[...]
    keys = _keys_of(x)
    tie = keys == kappa
    m_b = jnp.sum(tie.astype(jnp.int32))

    @pl.when(m_b == 0)
    def _simple():
        o_ref[...] = jnp.where(keys >= kappa, x, jnp.float32(0))

    @pl.when(m_b > 0)
    def _withties():
        r_iota = jax.lax.broadcasted_iota(jnp.int32, (rows, d), 0)
        c_iota = jax.lax.broadcasted_iota(jnp.int32, (rows, d), 1)
        flat = (r_iota + b * rows) * d + c_iota
        keep = (keys > kappa) | (tie & (flat <= cutoff))
        o_ref[...] = jnp.where(keep, x, jnp.float32(0))


def _slow_path(x, k_total):
    bsz, d = x.shape
    rows = bsz // _NBM
    kappa = pl.pallas_call(
        functools.partial(_bisect_kernel, k_total=k_total, nb=_NB, t=_T,
                          n_passes=_P),
        grid=(_P, _NB),
        in_specs=[pl.BlockSpec((bsz, d // _NB), lambda p, b: (0, b))],
        out_specs=pl.BlockSpec(memory_space=pltpu.SMEM),
        out_shape=jax.ShapeDtypeStruct((1,), jnp.int32),
        scratch_shapes=[pltpu.SMEM((_T,), jnp.int32),
                        pltpu.SMEM((2,), jnp.int32)],
    )(x)

    meta = pl.pallas_call(
        functools.partial(_ties_kernel, maxt=_MAXT, rows=rows, d=d),
        grid=(_NBM,),
        in_specs=[pl.BlockSpec(memory_space=pltpu.SMEM),
                  pl.BlockSpec((rows, d), lambda b: (b, 0))],
        out_specs=pl.BlockSpec(memory_space=pltpu.SMEM),
        out_shape=jax.ShapeDtypeStruct((_MAXT + 2,), jnp.int32),
    )(kappa, x)

    return pl.pallas_call(
        functools.partial(_mask_kernel, k_total=k_total, maxt=_MAXT, rows=rows,
                          d=d),
        grid=(_NBM,),
        in_specs=[pl.BlockSpec(memory_space=pltpu.SMEM),
                  pl.BlockSpec(memory_space=pltpu.SMEM),
                  pl.BlockSpec((rows, d), lambda b: (b, 0))],
        out_specs=pl.BlockSpec((rows, d), lambda b: (b, 0)),
        out_shape=jax.ShapeDtypeStruct((bsz, d), x.dtype),
    )(kappa, meta, x)


@jax.jit
def kernel(x):
    bsz, d = x.shape
    n = bsz * d
    k_total = min(_TOP_K * bsz, n)
    nchunk = n // (_NW * _CHUNK)

    mesh = plsc.VectorSubcoreMesh(core_axis_name="c", subcore_axis_name="s")
    compact = pl.kernel(
        functools.partial(_sc_compact_kernel, nchunk=nchunk),
        out_type=(jax.ShapeDtypeStruct((_NW * _TCAP,), jnp.float32),
                  jax.ShapeDtypeStruct((_NW * _TCAP,), jnp.int32),
                  jax.ShapeDtypeStruct((_NW * _LANES,), jnp.int32)),
        mesh=mesh,
        scratch_types=(pltpu.VMEM((1, _CHUNK), jnp.float32),
                       pltpu.VMEM((1, _CHUNK), jnp.float32),
                       pltpu.VMEM((1, _CHUNK), jnp.float32),
                       pltpu.VMEM((_TCAP,), jnp.float32),
                       pltpu.VMEM((_TCAP,), jnp.int32),
                       pltpu.VMEM((_LANES,), jnp.int32),
                       pltpu.SemaphoreType.DMA,
                       pltpu.SemaphoreType.DMA,
                       pltpu.SemaphoreType.DMA),
        compiler_params=pltpu.CompilerParams(needs_layout_passes=False),
    )
    cval, cidx, ccnt = compact(x)

    meta = pl.pallas_call(
        functools.partial(_select_kernel, k_total=k_total),
        grid=(1,),
        in_specs=[pl.BlockSpec((_NW * _LANES, _PCAP), lambda i: (0, 0)),
                  pl.BlockSpec((_NW * _LANES, _PCAP), lambda i: (0, 0)),
                  pl.BlockSpec((4, 128), lambda i: (0, 0))],
        out_specs=pl.BlockSpec(memory_space=pltpu.SMEM),
        out_shape=jax.ShapeDtypeStruct((4,), jnp.int32),
    )(cval.reshape(_NW * _LANES, _LCAP), cidx.reshape(_NW * _LANES, _LCAP),
      ccnt.reshape(4, 128))

    rows = bsz // _NBM

    def _fast(x, meta):
        return pl.pallas_call(
            functools.partial(_fastmask_kernel, rows=rows, d=d),
            grid=(_NBM,),
            in_specs=[pl.BlockSpec(memory_space=pltpu.SMEM),
                      pl.BlockSpec((rows, d), lambda b: (b, 0))],
            out_specs=pl.BlockSpec((rows, d), lambda b: (b, 0)),
            out_shape=jax.ShapeDtypeStruct((bsz, d), x.dtype),
        )(meta, x)

    return jax.lax.cond(meta[0] == 1,
                        lambda: _fast(x, meta),
                        lambda: _slow_path(x, k_total))


# speedup vs baseline: 2.7347x; 1.0774x over previous
import functools

import jax
import jax.numpy as jnp
import numpy as np
from jax.experimental import pallas as pl
from jax.experimental.pallas import tpu as pltpu
from jax.experimental.pallas import tpu_sc as plsc

_TOP_K = 32
_IMAX = np.int32(2**31 - 1)

_THRESH = np.float32(2.25)
_KST = np.int32(np.float32(2.25).view(np.int32))
_NC = 2
_NS = 16
_NW = _NC * _NS
_LANES = 16
_CHUNK = 16384
_NBUF = 3
_UNROLL = 4
_LCAP = 2048
_PCAP = 512
_TCAP = _LANES * _LCAP
_T2 = 8

_T = 16
_NB = 16
_NBM = 16
_MAXT = 16


def _num_passes(width, t):
    w = width
    p = 0
    while w > 0:
        w //= t + 1
        p += 1
    return p


_P = _num_passes(2**32 - 1, _T)
_R2 = _num_passes(int(_IMAX) - int(_KST), _T2)


def _keys_of(x):
    b = jax.lax.bitcast_convert_type(x, jnp.int32)
    flip = jax.lax.shift_right_arithmetic(b, 31) & jnp.int32(2**31 - 1)
    return b ^ flip


def _sc_compact_kernel(x_hbm, val_hbm, idx_hbm, cnt_hbm,
                       vbuf_a, vbuf_b, vbuf_c, val_l, idx_l, cnt_l,
                       sem_a, sem_b, sem_c, *, nchunk):
    wid = jax.lax.axis_index("s") * _NC + jax.lax.axis_index("c")
    base = wid * (_CHUNK * nchunk)

    zero16 = jnp.zeros((_LANES,), jnp.float32)

    @plsc.parallel_loop(0, _TCAP // _LANES, unroll=8)
    def _zinit(i):
        val_l[pl.ds(i * _LANES, _LANES)] = zero16

    lane_base = jax.lax.iota(jnp.int32, _LANES) * _LCAP
    pos = jnp.zeros((_LANES,), jnp.int32)
    iv = jax.lax.iota(jnp.int32, _LANES) + base

    bufs = [vbuf_a, vbuf_b, vbuf_c]

    def _run_chunk(cur, pos, iv):
        def _inner(i, carry):
            pos, iv = carry
            v = bufs[cur][0, pl.ds(i * _LANES, _LANES)]
            m = v >= _THRESH
            tgt = lane_base + jnp.minimum(pos, _LCAP - 1)
            plsc.store_scatter(val_l, [tgt], v, mask=m)
            plsc.store_scatter(idx_l, [tgt], iv, mask=m)
            pos = pos + m.astype(jnp.int32)
            iv = iv + _LANES
            return pos, iv

        return plsc.parallel_loop(0, _CHUNK // _LANES, unroll=_UNROLL,
                                  carry=(pos, iv))(_inner)

    row0 = wid * nchunk
    sems = [sem_a, sem_b, sem_c]
    copies = [None] * _NBUF
    for c in range(min(_NBUF, nchunk)):
        copies[c] = pltpu.async_copy(
            x_hbm.at[pl.ds(row0 + c, 1), :], bufs[c], sems[c])
    for c in range(nchunk):
        cur = c % _NBUF
        copies[cur].wait()
        pos, iv = _run_chunk(cur, pos, iv)
        if c + _NBUF < nchunk:
            copies[cur] = pltpu.async_copy(
                x_hbm.at[pl.ds(row0 + c + _NBUF, 1), :],
                bufs[cur], sems[cur])

    cnt_l[...] = pos
    pltpu.sync_copy(val_l, val_hbm.at[pl.ds(wid * _TCAP, _TCAP)])
    pltpu.sync_copy(idx_l, idx_hbm.at[pl.ds(wid * _TCAP, _TCAP)])
    pltpu.sync_copy(cnt_l, cnt_hbm.at[pl.ds(wid * _LANES, _LANES)])


def _select_kernel(val_ref, idx_ref, cnt_ref, meta_ref, *, k_total):
    vb = jax.lax.bitcast_convert_type(val_ref[...], jnp.int32)
    cnt = cnt_ref[...]

    n_cand = jnp.sum(cnt)
    ok = jnp.all(cnt <= _PCAP) & (n_cand >= k_total)

    lo = jnp.int32(_KST)
    hi = jnp.int32(_IMAX)
    for _ in range(_R2):
        step = jax.lax.div(hi - lo, jnp.int32(_T2 + 1)) + jnp.int32(1)
        ts = [lo + step * jnp.int32(i + 1) - jnp.int32(1) for i in range(_T2)]
        cs = [jnp.sum((vb > t).astype(jnp.int32)) for t in ts]
        for i in range(_T2):
            lo = jnp.where(cs[i] >= k_total, ts[i] + jnp.int32(1), lo)
            hi = jnp.where(cs[i] < k_total, jnp.minimum(hi, ts[i]), hi)
    kappa = lo

    tie = vb == kappa
    m = jnp.sum(tie.astype(jnp.int32))
    c_ge = jnp.sum((vb >= kappa).astype(jnp.int32))
    r = k_total - (c_ge - m)
    ok = ok & (m <= _MAXT)

    idx = idx_ref[...]
    cand0 = jnp.where(tie, idx, _IMAX)
    cutoff = jnp.int32(-1)
    last = jnp.int32(-1)
    for j in range(_MAXT):
        nxt = jnp.min(jnp.where(cand0 > last, cand0, _IMAX))
        cutoff = jnp.where(jnp.int32(j) == r - 1, nxt, cutoff)
        last = nxt

    meta_ref[0] = jnp.where(ok, jnp.int32(1), jnp.int32(0))
    meta_ref[1] = kappa
    meta_ref[2] = cutoff
    meta_ref[3] = n_cand


def _fastmask_kernel(meta_ref, x_ref, o_ref, *, rows, d):
    b = pl.program_id(0)
    kappa = meta_ref[1]
    cutoff = meta_ref[2]
    x = x_ref[...]
    xb = jax.lax.bitcast_convert_type(x, jnp.int32)
    tie = xb == kappa
    m_b = jnp.sum(tie.astype(jnp.int32))

    @pl.when(m_b == 0)
    def _simple():
        o_ref[...] = jnp.where(xb >= kappa, x, jnp.float32(0))

    @pl.when(m_b > 0)
    def _withties():
        r_iota = jax.lax.broadcasted_iota(jnp.int32, (rows, d), 0)
        c_iota = jax.lax.broadcasted_iota(jnp.int32, (rows, d), 1)
        flat = (r_iota + b * rows) * d + c_iota
        keep = (xb > kappa) | (tie & (flat <= cutoff))
        o_ref[...] = jnp.where(keep, x, jnp.float32(0))


def _bisect_kernel(x_ref, kappa_ref, counts_ref, state_ref, *, k_total, nb, t,
                   n_passes):
    p = pl.program_id(0)
    b = pl.program_id(1)

    @pl.when((p == 0) & (b == 0))
    def _init():
        state_ref[0] = jnp.int32(-(2**31))
        state_ref[1] = jnp.int32(2**31 - 1)
        for i in range(t):
            counts_ref[i] = jnp.int32(0)

    lo = state_ref[0]
    hi = state_ref[1]
    step0 = jnp.int32(2**32 // (t + 1) + 1)
    stepg = jax.lax.div(hi - lo, jnp.int32(t + 1)) + jnp.int32(1)
    step = jnp.where(p == 0, step0, stepg)
    ts = [lo + step * jnp.int32(i + 1) - jnp.int32(1) for i in range(t)]

    keys = _keys_of(x_ref[...])
    for i in range(t):
        c = jnp.sum((keys > ts[i]).astype(jnp.int32))
        counts_ref[i] = counts_ref[i] + c

    @pl.when(b == nb - 1)
    def _update():
        new_lo = lo
        new_hi = hi
        for i in range(t):
            ci = counts_ref[i]
            new_lo = jnp.where(ci >= k_total, ts[i] + jnp.int32(1), new_lo)
            new_hi = jnp.where(ci < k_total, jnp.minimum(new_hi, ts[i]), new_hi)
        state_ref[0] = new_lo
        state_ref[1] = new_hi
        for i in range(t):
            counts_ref[i] = jnp.int32(0)

        @pl.when(p == n_passes - 1)
        def _fin():
            kappa_ref[0] = new_lo


def _ties_kernel(kappa_ref, x_ref, meta_ref, *, maxt, rows, d):
    b = pl.program_id(0)

    @pl.when(b == 0)
    def _init():
        for i in range(maxt + 2):
            meta_ref[i] = jnp.int32(0)

    kappa = kappa_ref[0]
    keys = _keys_of(x_ref[...])
    ge = keys >= kappa
    meta_ref[maxt] = meta_ref[maxt] + jnp.sum(ge.astype(jnp.int32))
    tie = keys == kappa
    m_b = jnp.sum(tie.astype(jnp.int32))

    @pl.when(m_b > 0)
    def _extract():
        r_iota = jax.lax.broadcasted_iota(jnp.int32, (rows, d), 0)
        c_iota = jax.lax.broadcasted_iota(jnp.int32, (rows, d), 1)
        flat = (r_iota + b * rows) * d + c_iota
        cand = jnp.where(tie, flat, _IMAX)
        last = jnp.int32(-1)
        for _ in range(maxt):
            nxt = jnp.min(jnp.where(cand > last, cand, _IMAX))
            found = nxt != _IMAX
            pos = meta_ref[maxt + 1]

            @pl.when(found & (pos < maxt))
            def _store():
                meta_ref[pos] = nxt

            meta_ref[maxt + 1] = jnp.where(found, pos + 1, pos)
            last = jnp.where(found, nxt, last)


def _mask_kernel(kappa_ref, meta_ref, x_ref, o_ref, *, k_total, maxt, rows, d):
    b = pl.program_id(0)
    kappa = kappa_ref[0]
    c_ge = meta_ref[maxt]
    n_ties = meta_ref[maxt + 1]
    c_gt = c_ge - n_ties
    r = k_total - c_gt
    ridx = jnp.clip(r - 1, 0, maxt - 1)
    cutoff = jnp.where(r <= 0, jnp.int32(-1), meta_ref[ridx])

    x = x_ref[...]
    keys = _keys_of(x)
    tie = keys == kappa
    m_b = jnp.sum(tie.astype(jnp.int32))

    @pl.when(m_b == 0)
    def _simple():
        o_ref[...] = jnp.where(keys >= kappa, x, jnp.float32(0))

    @pl.when(m_b > 0)
    def _withties():
        r_iota = jax.lax.broadcasted_iota(jnp.int32, (rows, d), 0)
        c_iota = jax.lax.broadcasted_iota(jnp.int32, (rows, d), 1)
        flat = (r_iota + b * rows) * d + c_iota
        keep = (keys > kappa) | (tie & (flat <= cutoff))
        o_ref[...] = jnp.where(keep, x, jnp.float32(0))


def _slow_path(x, k_total):
    bsz, d = x.shape
    rows = bsz // _NBM
    kappa = pl.pallas_call(
        functools.partial(_bisect_kernel, k_total=k_total, nb=_NB, t=_T,
                          n_passes=_P),
        grid=(_P, _NB),
        in_specs=[pl.BlockSpec((bsz, d // _NB), lambda p, b: (0, b))],
        out_specs=pl.BlockSpec(memory_space=pltpu.SMEM),
        out_shape=jax.ShapeDtypeStruct((1,), jnp.int32),
        scratch_shapes=[pltpu.SMEM((_T,), jnp.int32),
                        pltpu.SMEM((2,), jnp.int32)],
    )(x)

    meta = pl.pallas_call(
        functools.partial(_ties_kernel, maxt=_MAXT, rows=rows, d=d),
        grid=(_NBM,),
        in_specs=[pl.BlockSpec(memory_space=pltpu.SMEM),
                  pl.BlockSpec((rows, d), lambda b: (b, 0))],
        out_specs=pl.BlockSpec(memory_space=pltpu.SMEM),
        out_shape=jax.ShapeDtypeStruct((_MAXT + 2,), jnp.int32),
    )(kappa, x)

    return pl.pallas_call(
        functools.partial(_mask_kernel, k_total=k_total, maxt=_MAXT, rows=rows,
                          d=d),
        grid=(_NBM,),
        in_specs=[pl.BlockSpec(memory_space=pltpu.SMEM),
                  pl.BlockSpec(memory_space=pltpu.SMEM),
                  pl.BlockSpec((rows, d), lambda b: (b, 0))],
        out_specs=pl.BlockSpec((rows, d), lambda b: (b, 0)),
        out_shape=jax.ShapeDtypeStruct((bsz, d), x.dtype),
    )(kappa, meta, x)


@jax.jit
def kernel(x):
    bsz, d = x.shape
    n = bsz * d
    k_total = min(_TOP_K * bsz, n)
    nchunk = n // (_NW * _CHUNK)

    mesh = plsc.VectorSubcoreMesh(core_axis_name="c", subcore_axis_name="s")
    compact = pl.kernel(
        functools.partial(_sc_compact_kernel, nchunk=nchunk),
        out_type=(jax.ShapeDtypeStruct((_NW * _TCAP,), jnp.float32),
                  jax.ShapeDtypeStruct((_NW * _TCAP,), jnp.int32),
                  jax.ShapeDtypeStruct((_NW * _LANES,), jnp.int32)),
        mesh=mesh,
        scratch_types=(pltpu.VMEM((1, _CHUNK), jnp.float32),
                       pltpu.VMEM((1, _CHUNK), jnp.float32),
                       pltpu.VMEM((1, _CHUNK), jnp.float32),
                       pltpu.VMEM((_TCAP,), jnp.float32),
                       pltpu.VMEM((_TCAP,), jnp.int32),
                       pltpu.VMEM((_LANES,), jnp.int32),
                       pltpu.SemaphoreType.DMA,
                       pltpu.SemaphoreType.DMA,
                       pltpu.SemaphoreType.DMA),
        compiler_params=pltpu.CompilerParams(needs_layout_passes=False),
    )
    cval, cidx, ccnt = compact(x)

    meta = pl.pallas_call(
        functools.partial(_select_kernel, k_total=k_total),
        grid=(1,),
        in_specs=[pl.BlockSpec((_NW * _LANES, _PCAP), lambda i: (0, 0)),
                  pl.BlockSpec((_NW * _LANES, _PCAP), lambda i: (0, 0)),
                  pl.BlockSpec((4, 128), lambda i: (0, 0))],
        out_specs=pl.BlockSpec(memory_space=pltpu.SMEM),
        out_shape=jax.ShapeDtypeStruct((4,), jnp.int32),
    )(cval.reshape(_NW * _LANES, _LCAP), cidx.reshape(_NW * _LANES, _LCAP),
      ccnt.reshape(4, 128))

    rows = bsz // _NBM

    def _fast(x, meta):
        return pl.pallas_call(
            functools.partial(_fastmask_kernel, rows=rows, d=d),
            grid=(_NBM,),
            in_specs=[pl.BlockSpec(memory_space=pltpu.SMEM),
                      pl.BlockSpec((rows, d), lambda b: (b, 0))],
            out_specs=pl.BlockSpec((rows, d), lambda b: (b, 0)),
            out_shape=jax.ShapeDtypeStruct((bsz, d), x.dtype),
        )(meta, x)

    return jax.lax.cond(meta[0] == 1,
                        lambda: _fast(x, meta),
                        lambda: _slow_path(x, k_total))
